# baseline (speedup 1.0000x reference)
import functools

import jax
import jax.numpy as jnp
from jax import lax
from jax.experimental import pallas as pl
from jax.experimental.pallas import tpu as pltpu
from jax.experimental.pallas import tpu_sc as plsc

B = 8
N = 8192
C = 16
S = 512
NSAMP = 32
R2 = 0.2 * 0.2
CH_IN = 32
C1 = 32
C2 = 64
M = B * S * NSAMP
EPS = 1e-5


def _fps_body(xyzs_ref, nx_ref, ny_ref, nz_ref, dist_ref):
    nb = xyzs_ref.shape[0] // 3
    xyzs = xyzs_ref[...]
    xs = xyzs[0:nb]
    ys = xyzs[nb:2 * nb]
    zs = xyzs[2 * nb:3 * nb]
    dist_ref[...] = jnp.full((nb, N), 1e10, jnp.float32)
    nx_ref[...] = jnp.zeros((nb, S), jnp.float32)
    ny_ref[...] = jnp.zeros((nb, S), jnp.float32)
    nz_ref[...] = jnp.zeros((nb, S), jnp.float32)
    iota_n = lax.broadcasted_iota(jnp.int32, (nb, N), 1)
    iota_s = lax.broadcasted_iota(jnp.int32, (nb, S), 1)

    def step(i, far):
        ohf = (iota_n == far).astype(jnp.float32)
        oh3 = jnp.concatenate([ohf] * 3, axis=0)
        cs = jnp.sum(oh3 * xyzs, axis=1, keepdims=True)
        cx = cs[0:nb]
        cy = cs[nb:2 * nb]
        cz = cs[2 * nb:3 * nb]
        sel = iota_s == i
        nx_ref[...] += jnp.where(sel, cx, 0.0)
        ny_ref[...] += jnp.where(sel, cy, 0.0)
        nz_ref[...] += jnp.where(sel, cz, 0.0)
        dx = xs - cx
        dy = ys - cy
        dz = zs - cz
        d = dx * dx + dy * dy + dz * dz
        dist = jnp.minimum(dist_ref[...], d)
        dist_ref[...] = dist
        m = jnp.max(dist, axis=1, keepdims=True)
        far_n = jnp.min(jnp.where(dist == m, iota_n, N), axis=1, keepdims=True)
        return far_n

    lax.fori_loop(0, S, step, jnp.zeros((nb, 1), jnp.int32))


def _fps(xyzs):
    nb = xyzs.shape[0] // 3
    return pl.pallas_call(
        _fps_body,
        out_shape=[jax.ShapeDtypeStruct((nb, S), jnp.float32)] * 3,
        scratch_shapes=[pltpu.VMEM((nb, N), jnp.float32)],
    )(xyzs)


_NW = 32
_HB = 4
_WPB = _NW // _HB
_S_PER_W = S // _WPB
_NBLK = N // 16
_UNROLL = 4
_GRP = 8


def _bq_body(batch_off, xs_hbm, ys_hbm, zs_hbm, nx_hbm, ny_hbm, nz_hbm,
             table_hbm, out_hbm, xv, yv, zv, nxv, nyv, nzv, idxbuf, gidx,
             rows, sem):
    wid = lax.axis_index("s") * 2 + lax.axis_index("c")
    b = wid // _WPB
    s0 = (wid % _WPB) * _S_PER_W

    pltpu.sync_copy(xs_hbm.at[b], xv)
    pltpu.sync_copy(ys_hbm.at[b], yv)
    pltpu.sync_copy(zs_hbm.at[b], zv)
    pltpu.sync_copy(nx_hbm.at[b, pl.ds(s0, _S_PER_W)], nxv)
    pltpu.sync_copy(ny_hbm.at[b, pl.ds(s0, _S_PER_W)], nyv)
    pltpu.sync_copy(nz_hbm.at[b, pl.ds(s0, _S_PER_W)], nzv)

    lane = lax.iota(jnp.int32, 16)
    goff = (batch_off + b) * N

    def group(g, carry):
        for jj in range(_GRP):
            j = g * _GRP + jj
            jo = (j // 16) * 16
            jv = jnp.full((16,), j % 16, jnp.int32)
            cx = nxv[pl.ds(jo, 16)][jv]
            cy = nyv[pl.ds(jo, 16)][jv]
            cz = nzv[pl.ds(jo, 16)][jv]

            def cond(st):
                nb, cnt = st
                return jnp.logical_and(cnt < NSAMP, nb < _NBLK)

            def body(st):
                nb, cnt = st
                cntv = jnp.full((16,), cnt, jnp.int32)
                for u in range(_UNROLL):
                    base = (nb + u) * 16
                    dx = xv[pl.ds(base, 16)] - cx
                    dy = yv[pl.ds(base, 16)] - cy
                    dz = zv[pl.ds(base, 16)] - cz
                    d = dx * dx + dy * dy + dz * dz
                    mask = d <= R2
                    c = plsc.cumsum(mask.astype(jnp.int32))
                    plsc.store_scatter(idxbuf, [cntv + c - 1], lane + base,
                                       mask=mask)
                    cntv = cntv + plsc.all_reduce_population_count(mask)
                return nb + _UNROLL, jnp.max(cntv)

            _, cnt = lax.while_loop(cond, body, (jnp.int32(0), jnp.int32(0)))
            m = jnp.minimum(cnt, NSAMP)
            lo_raw = idxbuf[pl.ds(0, 16)]
            first = lo_raw[jnp.zeros((16,), jnp.int32)]
            lo = jnp.where(lane < m, lo_raw, first)
            hi = jnp.where(lane + 16 < m, idxbuf[pl.ds(16, 16)], first)
            gidx[pl.ds(jj * NSAMP, 16)] = lo + goff
            gidx[pl.ds(jj * NSAMP + 16, 16)] = hi + goff
        pltpu.async_copy(table_hbm.at[gidx], rows, sem).wait()
        obase = (b * S + s0 + g * _GRP) * NSAMP
        pltpu.sync_copy(rows, out_hbm.at[pl.ds(obase, _GRP * NSAMP)])
        return carry

    lax.fori_loop(0, _S_PER_W // _GRP, group, jnp.int32(0))


@functools.cache
def _ball_gather(batch_off):
    mesh = plsc.VectorSubcoreMesh(core_axis_name="c", subcore_axis_name="s")
    return pl.kernel(
        functools.partial(_bq_body, batch_off),
        out_type=jax.ShapeDtypeStruct((_HB * S * NSAMP, CH_IN), jnp.float32),
        mesh=mesh,
        compiler_params=pltpu.CompilerParams(needs_layout_passes=False,
                                             use_tc_tiling_on_sc=False),
        scratch_types=[
            pltpu.VMEM((N,), jnp.float32),
            pltpu.VMEM((N,), jnp.float32),
            pltpu.VMEM((N,), jnp.float32),
            pltpu.VMEM((_S_PER_W,), jnp.float32),
            pltpu.VMEM((_S_PER_W,), jnp.float32),
            pltpu.VMEM((_S_PER_W,), jnp.float32),
            pltpu.VMEM((31 + 16 * _UNROLL + 1,), jnp.int32),
            pltpu.VMEM((_GRP * NSAMP,), jnp.int32),
            pltpu.VMEM((_GRP * NSAMP, CH_IN), jnp.float32),
            pltpu.SemaphoreType.DMA,
        ],
    )


_CHUNK = 8192
_GCHUNK = _CHUNK // NSAMP
_NCHUNK = M // _CHUNK
_HCH = _NCHUNK // 2


_CDIM = (((1,), (1,)), ((), ()))


def _mlp_body(ga_ref, gb_ref, nxp_ref, w1_ref, b1_ref, g1_ref, be1_ref,
              w2_ref, b2_ref, g2_ref, be2_ref, out_ref,
              s1_ref, q1_ref, s2_ref, q2_ref):
    i = pl.program_id(0)
    phase = i // _NCHUNK
    c = i % _NCHUNK

    def pre1():
        w1 = w1_ref[...]
        bc = lax.dot_general(nxp_ref[...], w1, _CDIM,
                             preferred_element_type=jnp.float32)
        g = jnp.where(c < _HCH, ga_ref[...], gb_ref[...])
        p = lax.dot_general(g, w1, _CDIM,
                            preferred_element_type=jnp.float32) + b1_ref[...]
        return (p.reshape(_GCHUNK, NSAMP, C1) - bc[:, None, :]).reshape(
            _CHUNK, C1)

    def hval():
        mu = s1_ref[...] / M
        var = q1_ref[...] / M - mu * mu
        sc = g1_ref[...] * lax.rsqrt(var + EPS)
        sh = be1_ref[...] - mu * sc
        return jnp.maximum(pre1() * sc + sh, 0.0)

    @pl.when(phase == 0)
    def _():
        p = pre1()

        @pl.when(i == 0)
        def _():
            s1_ref[...] = jnp.zeros((1, C1), jnp.float32)
            q1_ref[...] = jnp.zeros((1, C1), jnp.float32)

        s1_ref[...] += jnp.sum(p, axis=0, keepdims=True)
        q1_ref[...] += jnp.sum(p * p, axis=0, keepdims=True)

    @pl.when(phase == 1)
    def _():
        p2 = lax.dot_general(hval(), w2_ref[...], _CDIM,
                             preferred_element_type=jnp.float32) + b2_ref[...]

        @pl.when(i == _NCHUNK)
        def _():
            s2_ref[...] = jnp.zeros((1, C2), jnp.float32)
            q2_ref[...] = jnp.zeros((1, C2), jnp.float32)

        s2_ref[...] += jnp.sum(p2, axis=0, keepdims=True)
        q2_ref[...] += jnp.sum(p2 * p2, axis=0, keepdims=True)

    @pl.when(phase == 2)
    def _():
        mu = s2_ref[...] / M
        var = q2_ref[...] / M - mu * mu
        sc = g2_ref[...] * lax.rsqrt(var + EPS)
        sh = be2_ref[...] - mu * sc
        p2 = lax.dot_general(hval(), w2_ref[...], _CDIM,
                             preferred_element_type=jnp.float32) + b2_ref[...]
        y = jnp.maximum(p2 * sc + sh, 0.0)
        out_ref[...] = jnp.max(y.reshape(_GCHUNK, NSAMP, C2), axis=1)


def _mlp(ga, gb, nxp, w1, b1, g1, be1, w2, b2, g2, be2):
    nc = _NCHUNK

    def gamap(i):
        c = i % nc
        return (jnp.where(c < _HCH, c, 0), 0)

    def gbmap(i):
        c = i % nc
        return (jnp.where(c >= _HCH, c - _HCH, 0), 0)

    def nmap(i):
        return (i % nc, 0)

    def omap(i):
        return (jnp.where(i >= 2 * nc, i - 2 * nc, 0), 0)

    def cmap(i):
        return (0, 0)

    return pl.pallas_call(
        _mlp_body,
        grid=(3 * nc,),
        in_specs=[pl.BlockSpec((_CHUNK, CH_IN), gamap),
                  pl.BlockSpec((_CHUNK, CH_IN), gbmap),
                  pl.BlockSpec((_GCHUNK, CH_IN), nmap),
                  pl.BlockSpec((C1, CH_IN), cmap),
                  pl.BlockSpec((1, C1), cmap),
                  pl.BlockSpec((1, C1), cmap),
                  pl.BlockSpec((1, C1), cmap),
                  pl.BlockSpec((C2, C1), cmap),
                  pl.BlockSpec((1, C2), cmap),
                  pl.BlockSpec((1, C2), cmap),
                  pl.BlockSpec((1, C2), cmap)],
        out_specs=pl.BlockSpec((_GCHUNK, C2), omap),
        out_shape=jax.ShapeDtypeStruct((B * S, C2), jnp.float32),
        scratch_shapes=[
            pltpu.VMEM((1, C1), jnp.float32),
            pltpu.VMEM((1, C1), jnp.float32),
            pltpu.VMEM((1, C2), jnp.float32),
            pltpu.VMEM((1, C2), jnp.float32),
        ],
    )(ga, gb, nxp, w1, b1, g1, be1, w2, b2, g2, be2)


def kernel(xyz, features, W1, b1, g1, be1, W2, b2, g2, be2):
    table = jnp.concatenate(
        [xyz, jnp.transpose(features, (0, 2, 1)),
         jnp.zeros((B, N, CH_IN - 3 - C), jnp.float32)], axis=-1,
    ).reshape(B * N, CH_IN)

    planes, gs = [], []
    for h in range(B // _HB):
        xyzs_h = jnp.transpose(xyz[h * _HB:(h + 1) * _HB],
                               (2, 0, 1)).reshape(3 * _HB, N)
        nx, ny, nz = _fps(xyzs_h)
        g_h = _ball_gather(h * _HB)(
            xyzs_h[0:_HB], xyzs_h[_HB:2 * _HB], xyzs_h[2 * _HB:3 * _HB],
            nx, ny, nz, table)
        planes.append((nx, ny, nz))
        gs.append(g_h)

    nx = jnp.concatenate([p[0] for p in planes], axis=0)
    ny = jnp.concatenate([p[1] for p in planes], axis=0)
    nz = jnp.concatenate([p[2] for p in planes], axis=0)
    new_xyz = jnp.stack([nx, ny, nz], axis=-1)

    nxp = jnp.concatenate(
        [new_xyz.reshape(B * S, 3),
         jnp.zeros((B * S, CH_IN - 3), jnp.float32)], axis=-1)
    w1e = jnp.concatenate(
        [W1, jnp.zeros((C1, CH_IN - W1.shape[1]), jnp.float32)], axis=-1)

    out = _mlp(gs[0], gs[1], nxp, w1e, b1.reshape(1, C1), g1.reshape(1, C1),
               be1.reshape(1, C1), W2, b2.reshape(1, C2),
               g2.reshape(1, C2), be2.reshape(1, C2))

    new_features = jnp.transpose(out.reshape(B, S, C2), (0, 2, 1))
    return new_xyz, new_features


# speedup vs baseline: 1.1465x; 1.1465x over previous
import functools

import jax
import jax.numpy as jnp
from jax import lax
from jax.experimental import pallas as pl
from jax.experimental.pallas import tpu as pltpu
from jax.experimental.pallas import tpu_sc as plsc

B = 8
N = 8192
C = 16
S = 512
NSAMP = 32
R2 = 0.2 * 0.2
CH_IN = 32
C1 = 32
C2 = 64
M = B * S * NSAMP
EPS = 1e-5


def _fps_body(xyzs_ref, nx_ref, ny_ref, nz_ref, dist_ref):
    xyzs = xyzs_ref[...]
    xs = xyzs[0:B]
    ys = xyzs[B:2 * B]
    zs = xyzs[2 * B:3 * B]
    dist_ref[...] = jnp.full((B, N), 1e10, jnp.float32)
    nx_ref[...] = jnp.zeros((B, S), jnp.float32)
    ny_ref[...] = jnp.zeros((B, S), jnp.float32)
    nz_ref[...] = jnp.zeros((B, S), jnp.float32)
    iota_n = lax.broadcasted_iota(jnp.int32, (B, N), 1)
    iota_s = lax.broadcasted_iota(jnp.int32, (B, S), 1)

    def step(i, far):
        ohf = (iota_n == far).astype(jnp.float32)
        cx = jnp.sum(ohf * xs, axis=1, keepdims=True)
        cy = jnp.sum(ohf * ys, axis=1, keepdims=True)
        cz = jnp.sum(ohf * zs, axis=1, keepdims=True)
        sel = iota_s == i
        nx_ref[...] += jnp.where(sel, cx, 0.0)
        ny_ref[...] += jnp.where(sel, cy, 0.0)
        nz_ref[...] += jnp.where(sel, cz, 0.0)
        dx = xs - cx
        dy = ys - cy
        dz = zs - cz
        d = dx * dx + dy * dy + dz * dz
        dist = jnp.minimum(dist_ref[...], d)
        dist_ref[...] = dist
        m = jnp.max(dist, axis=1, keepdims=True)
        far_n = jnp.min(jnp.where(dist == m, iota_n, N), axis=1, keepdims=True)
        return far_n

    lax.fori_loop(0, S, step, jnp.zeros((B, 1), jnp.int32))


def _fps(xyzs):
    return pl.pallas_call(
        _fps_body,
        out_shape=[jax.ShapeDtypeStruct((B, S), jnp.float32)] * 3,
        scratch_shapes=[pltpu.VMEM((B, N), jnp.float32)],
    )(xyzs)


_NW = 32
_S_PER_W = S // (_NW // B)
_NBLK = N // 16
_UNROLL = 4
_GRP = 8


def _bq_body(xs_hbm, ys_hbm, zs_hbm, nx_hbm, ny_hbm, nz_hbm, table_hbm,
             out_hbm, xv, yv, zv, nxv, nyv, nzv, idxbuf, gidx, rows, sem):
    wid = lax.axis_index("s") * 2 + lax.axis_index("c")
    b = wid // 4
    s0 = (wid % 4) * _S_PER_W

    pltpu.sync_copy(xs_hbm.at[b], xv)
    pltpu.sync_copy(ys_hbm.at[b], yv)
    pltpu.sync_copy(zs_hbm.at[b], zv)
    pltpu.sync_copy(nx_hbm.at[b, pl.ds(s0, _S_PER_W)], nxv)
    pltpu.sync_copy(ny_hbm.at[b, pl.ds(s0, _S_PER_W)], nyv)
    pltpu.sync_copy(nz_hbm.at[b, pl.ds(s0, _S_PER_W)], nzv)

    lane = lax.iota(jnp.int32, 16)
    goff = b * N

    def group(g, carry):
        for jj in range(_GRP):
            j = g * _GRP + jj
            jo = (j // 16) * 16
            jv = jnp.full((16,), j % 16, jnp.int32)
            cx = nxv[pl.ds(jo, 16)][jv]
            cy = nyv[pl.ds(jo, 16)][jv]
            cz = nzv[pl.ds(jo, 16)][jv]

            def cond(st):
                nb, cnt = st
                return jnp.logical_and(cnt < NSAMP, nb < _NBLK)

            def body(st):
                nb, cnt = st
                cntv = jnp.full((16,), cnt, jnp.int32)
                for u in range(_UNROLL):
                    base = (nb + u) * 16
                    dx = xv[pl.ds(base, 16)] - cx
                    dy = yv[pl.ds(base, 16)] - cy
                    dz = zv[pl.ds(base, 16)] - cz
                    d = dx * dx + dy * dy + dz * dz
                    mask = d <= R2
                    c = plsc.cumsum(mask.astype(jnp.int32))
                    plsc.store_scatter(idxbuf, [cntv + c - 1], lane + base,
                                       mask=mask)
                    cntv = cntv + plsc.all_reduce_population_count(mask)
                return nb + _UNROLL, jnp.max(cntv)

            _, cnt = lax.while_loop(cond, body, (jnp.int32(0), jnp.int32(0)))
            m = jnp.minimum(cnt, NSAMP)
            lo_raw = idxbuf[pl.ds(0, 16)]
            first = lo_raw[jnp.zeros((16,), jnp.int32)]
            lo = jnp.where(lane < m, lo_raw, first)
            hi = jnp.where(lane + 16 < m, idxbuf[pl.ds(16, 16)], first)
            gidx[pl.ds(jj * NSAMP, 16)] = lo + goff
            gidx[pl.ds(jj * NSAMP + 16, 16)] = hi + goff
        pltpu.async_copy(table_hbm.at[gidx], rows, sem).wait()
        obase = (b * S + s0 + g * _GRP) * NSAMP
        pltpu.sync_copy(rows, out_hbm.at[pl.ds(obase, _GRP * NSAMP)])
        return carry

    lax.fori_loop(0, _S_PER_W // _GRP, group, jnp.int32(0))


@functools.cache
def _ball_gather():
    mesh = plsc.VectorSubcoreMesh(core_axis_name="c", subcore_axis_name="s")
    return pl.kernel(
        _bq_body,
        out_type=jax.ShapeDtypeStruct((M, CH_IN), jnp.float32),
        mesh=mesh,
        compiler_params=pltpu.CompilerParams(needs_layout_passes=False,
                                             use_tc_tiling_on_sc=False),
        scratch_types=[
            pltpu.VMEM((N,), jnp.float32),
            pltpu.VMEM((N,), jnp.float32),
            pltpu.VMEM((N,), jnp.float32),
            pltpu.VMEM((_S_PER_W,), jnp.float32),
            pltpu.VMEM((_S_PER_W,), jnp.float32),
            pltpu.VMEM((_S_PER_W,), jnp.float32),
            pltpu.VMEM((31 + 16 * _UNROLL + 1,), jnp.int32),
            pltpu.VMEM((_GRP * NSAMP,), jnp.int32),
            pltpu.VMEM((_GRP * NSAMP, CH_IN), jnp.float32),
            pltpu.SemaphoreType.DMA,
        ],
    )


_CHUNK = 8192
_GCHUNK = _CHUNK // NSAMP
_NCHUNK = M // _CHUNK


_CDIM = (((1,), (1,)), ((), ()))


def _mlp_body(g_ref, nxp_ref, w1_ref, b1_ref, g1_ref, be1_ref,
              w2_ref, b2_ref, g2_ref, be2_ref, out_ref,
              s1_ref, q1_ref, s2_ref, q2_ref):
    i = pl.program_id(0)
    phase = i // _NCHUNK

    def pre1():
        w1 = w1_ref[...]
        bc = lax.dot_general(nxp_ref[...], w1, _CDIM,
                             preferred_element_type=jnp.float32)
        p = lax.dot_general(g_ref[...], w1, _CDIM,
                            preferred_element_type=jnp.float32) + b1_ref[...]
        return (p.reshape(_GCHUNK, NSAMP, C1) - bc[:, None, :]).reshape(
            _CHUNK, C1)

    def hval():
        mu = s1_ref[...] / M
        var = q1_ref[...] / M - mu * mu
        sc = g1_ref[...] * lax.rsqrt(var + EPS)
        sh = be1_ref[...] - mu * sc
        return jnp.maximum(pre1() * sc + sh, 0.0)

    @pl.when(phase == 0)
    def _():
        p = pre1()

        @pl.when(i == 0)
        def _():
            s1_ref[...] = jnp.zeros((1, C1), jnp.float32)
            q1_ref[...] = jnp.zeros((1, C1), jnp.float32)

        s1_ref[...] += jnp.sum(p, axis=0, keepdims=True)
        q1_ref[...] += jnp.sum(p * p, axis=0, keepdims=True)

    @pl.when(phase == 1)
    def _():
        p2 = lax.dot_general(hval(), w2_ref[...], _CDIM,
                             preferred_element_type=jnp.float32) + b2_ref[...]

        @pl.when(i == _NCHUNK)
        def _():
            s2_ref[...] = jnp.zeros((1, C2), jnp.float32)
            q2_ref[...] = jnp.zeros((1, C2), jnp.float32)

        s2_ref[...] += jnp.sum(p2, axis=0, keepdims=True)
        q2_ref[...] += jnp.sum(p2 * p2, axis=0, keepdims=True)

    @pl.when(phase == 2)
    def _():
        mu = s2_ref[...] / M
        var = q2_ref[...] / M - mu * mu
        sc = g2_ref[...] * lax.rsqrt(var + EPS)
        sh = be2_ref[...] - mu * sc
        p2 = lax.dot_general(hval(), w2_ref[...], _CDIM,
                             preferred_element_type=jnp.float32) + b2_ref[...]
        y = jnp.maximum(p2 * sc + sh, 0.0)
        out_ref[...] = jnp.max(y.reshape(_GCHUNK, NSAMP, C2), axis=1)


def _mlp(g, nxp, w1, b1, g1, be1, w2, b2, g2, be2):
    nc = _NCHUNK

    def gmap(i):
        return (i % nc, 0)

    def omap(i):
        return (jnp.where(i >= 2 * nc, i - 2 * nc, 0), 0)

    def cmap(i):
        return (0, 0)

    return pl.pallas_call(
        _mlp_body,
        grid=(3 * nc,),
        in_specs=[pl.BlockSpec((_CHUNK, CH_IN), gmap),
                  pl.BlockSpec((_GCHUNK, CH_IN), gmap),
                  pl.BlockSpec((C1, CH_IN), cmap),
                  pl.BlockSpec((1, C1), cmap),
                  pl.BlockSpec((1, C1), cmap),
                  pl.BlockSpec((1, C1), cmap),
                  pl.BlockSpec((C2, C1), cmap),
                  pl.BlockSpec((1, C2), cmap),
                  pl.BlockSpec((1, C2), cmap),
                  pl.BlockSpec((1, C2), cmap)],
        out_specs=pl.BlockSpec((_GCHUNK, C2), omap),
        out_shape=jax.ShapeDtypeStruct((B * S, C2), jnp.float32),
        scratch_shapes=[
            pltpu.VMEM((1, C1), jnp.float32),
            pltpu.VMEM((1, C1), jnp.float32),
            pltpu.VMEM((1, C2), jnp.float32),
            pltpu.VMEM((1, C2), jnp.float32),
        ],
    )(g, nxp, w1, b1, g1, be1, w2, b2, g2, be2)


def kernel(xyz, features, W1, b1, g1, be1, W2, b2, g2, be2):
    xyzs = jnp.transpose(xyz, (2, 0, 1)).reshape(3 * B, N)
    xs = xyzs[0:B]
    ys = xyzs[B:2 * B]
    zs = xyzs[2 * B:3 * B]

    nx, ny, nz = _fps(xyzs)
    new_xyz = jnp.stack([nx, ny, nz], axis=-1)

    table = jnp.concatenate(
        [xyz, jnp.transpose(features, (0, 2, 1)),
         jnp.zeros((B, N, CH_IN - 3 - C), jnp.float32)], axis=-1,
    ).reshape(B * N, CH_IN)

    g = _ball_gather()(xs, ys, zs, nx, ny, nz, table)

    nxp = jnp.concatenate(
        [new_xyz.reshape(B * S, 3),
         jnp.zeros((B * S, CH_IN - 3), jnp.float32)], axis=-1)
    w1e = jnp.concatenate(
        [W1, jnp.zeros((C1, CH_IN - W1.shape[1]), jnp.float32)], axis=-1)

    out = _mlp(g, nxp, w1e, b1.reshape(1, C1), g1.reshape(1, C1),
               be1.reshape(1, C1), W2, b2.reshape(1, C2),
               g2.reshape(1, C2), be2.reshape(1, C2))

    new_features = jnp.transpose(out.reshape(B, S, C2), (0, 2, 1))
    return new_xyz, new_features


# speedup vs baseline: 1.1654x; 1.0165x over previous
import functools

import jax
import jax.numpy as jnp
from jax import lax
from jax.experimental import pallas as pl
from jax.experimental.pallas import tpu as pltpu
from jax.experimental.pallas import tpu_sc as plsc

B = 8
N = 8192
C = 16
S = 512
NSAMP = 32
R2 = 0.2 * 0.2
CH_IN = 32
C1 = 32
C2 = 64
M = B * S * NSAMP
EPS = 1e-5


def _fps_body(xyzs_ref, nx_ref, ny_ref, nz_ref, dist_ref):
    xyzs = xyzs_ref[...]
    xs = xyzs[0:B]
    ys = xyzs[B:2 * B]
    zs = xyzs[2 * B:3 * B]
    dist_ref[...] = jnp.full((B, N), 1e10, jnp.float32)
    nx_ref[...] = jnp.zeros((B, S), jnp.float32)
    ny_ref[...] = jnp.zeros((B, S), jnp.float32)
    nz_ref[...] = jnp.zeros((B, S), jnp.float32)
    iota_n = lax.broadcasted_iota(jnp.int32, (B, N), 1)
    iota_s = lax.broadcasted_iota(jnp.int32, (B, S), 1)

    def step(i, far):
        ohf = (iota_n == far).astype(jnp.float32)
        cx = jnp.sum(ohf * xs, axis=1, keepdims=True)
        cy = jnp.sum(ohf * ys, axis=1, keepdims=True)
        cz = jnp.sum(ohf * zs, axis=1, keepdims=True)
        sel = iota_s == i
        nx_ref[...] += jnp.where(sel, cx, 0.0)
        ny_ref[...] += jnp.where(sel, cy, 0.0)
        nz_ref[...] += jnp.where(sel, cz, 0.0)
        dx = xs - cx
        dy = ys - cy
        dz = zs - cz
        d = dx * dx + dy * dy + dz * dz
        dist = jnp.minimum(dist_ref[...], d)
        dist_ref[...] = dist
        m = jnp.max(dist, axis=1, keepdims=True)
        far_n = jnp.min(jnp.where(dist == m, iota_n, N), axis=1, keepdims=True)
        return far_n

    lax.fori_loop(0, S, step, jnp.zeros((B, 1), jnp.int32))


def _fps(xyzs):
    return pl.pallas_call(
        _fps_body,
        out_shape=[jax.ShapeDtypeStruct((B, S), jnp.float32)] * 3,
        scratch_shapes=[pltpu.VMEM((B, N), jnp.float32)],
    )(xyzs)


_NW = 32
_S_PER_W = S // (_NW // B)
_NBLK = N // 16
_UNROLL = 8
_GRP = 8


def _bq_body(xs_hbm, ys_hbm, zs_hbm, nx_hbm, ny_hbm, nz_hbm, table_hbm,
             out_hbm, xv, yv, zv, nxv, nyv, nzv, idxbuf, gidx, rows, sem):
    wid = lax.axis_index("s") * 2 + lax.axis_index("c")
    b = wid // 4
    s0 = (wid % 4) * _S_PER_W

    pltpu.sync_copy(xs_hbm.at[b], xv)
    pltpu.sync_copy(ys_hbm.at[b], yv)
    pltpu.sync_copy(zs_hbm.at[b], zv)
    pltpu.sync_copy(nx_hbm.at[b, pl.ds(s0, _S_PER_W)], nxv)
    pltpu.sync_copy(ny_hbm.at[b, pl.ds(s0, _S_PER_W)], nyv)
    pltpu.sync_copy(nz_hbm.at[b, pl.ds(s0, _S_PER_W)], nzv)

    lane = lax.iota(jnp.int32, 16)
    goff = b * N

    def group(g, carry):
        for jj in range(_GRP):
            j = g * _GRP + jj
            jo = (j // 16) * 16
            jv = jnp.full((16,), j % 16, jnp.int32)
            cx = nxv[pl.ds(jo, 16)][jv]
            cy = nyv[pl.ds(jo, 16)][jv]
            cz = nzv[pl.ds(jo, 16)][jv]

            def cond(st):
                nb, cnt = st
                return jnp.logical_and(cnt < NSAMP, nb < _NBLK)

            def body(st):
                nb, cnt = st
                cntv = jnp.full((16,), cnt, jnp.int32)
                for u in range(_UNROLL):
                    base = (nb + u) * 16
                    dx = xv[pl.ds(base, 16)] - cx
                    dy = yv[pl.ds(base, 16)] - cy
                    dz = zv[pl.ds(base, 16)] - cz
                    d = dx * dx + dy * dy + dz * dz
                    mask = d <= R2
                    c = plsc.cumsum(mask.astype(jnp.int32))
                    plsc.store_scatter(idxbuf, [cntv + c - 1], lane + base,
                                       mask=mask)
                    cntv = cntv + plsc.all_reduce_population_count(mask)
                return nb + _UNROLL, jnp.max(cntv)

            _, cnt = lax.while_loop(cond, body, (jnp.int32(0), jnp.int32(0)))
            m = jnp.minimum(cnt, NSAMP)
            lo_raw = idxbuf[pl.ds(0, 16)]
            first = lo_raw[jnp.zeros((16,), jnp.int32)]
            lo = jnp.where(lane < m, lo_raw, first)
            hi = jnp.where(lane + 16 < m, idxbuf[pl.ds(16, 16)], first)
            gidx[pl.ds(jj * NSAMP, 16)] = lo + goff
            gidx[pl.ds(jj * NSAMP + 16, 16)] = hi + goff
        pltpu.async_copy(table_hbm.at[gidx], rows, sem).wait()
        obase = (b * S + s0 + g * _GRP) * NSAMP
        pltpu.sync_copy(rows, out_hbm.at[pl.ds(obase, _GRP * NSAMP)])
        return carry

    lax.fori_loop(0, _S_PER_W // _GRP, group, jnp.int32(0))


@functools.cache
def _ball_gather():
    mesh = plsc.VectorSubcoreMesh(core_axis_name="c", subcore_axis_name="s")
    return pl.kernel(
        _bq_body,
        out_type=jax.ShapeDtypeStruct((M, CH_IN), jnp.float32),
        mesh=mesh,
        compiler_params=pltpu.CompilerParams(needs_layout_passes=False,
                                             use_tc_tiling_on_sc=False),
        scratch_types=[
            pltpu.VMEM((N,), jnp.float32),
            pltpu.VMEM((N,), jnp.float32),
            pltpu.VMEM((N,), jnp.float32),
            pltpu.VMEM((_S_PER_W,), jnp.float32),
            pltpu.VMEM((_S_PER_W,), jnp.float32),
            pltpu.VMEM((_S_PER_W,), jnp.float32),
            pltpu.VMEM((31 + 16 * _UNROLL + 1,), jnp.int32),
            pltpu.VMEM((_GRP * NSAMP,), jnp.int32),
            pltpu.VMEM((_GRP * NSAMP, CH_IN), jnp.float32),
            pltpu.SemaphoreType.DMA,
        ],
    )


_CHUNK = 8192
_GCHUNK = _CHUNK // NSAMP
_NCHUNK = M // _CHUNK


_CDIM = (((1,), (1,)), ((), ()))


def _mlp_body(g_ref, nxp_ref, w1_ref, b1_ref, g1_ref, be1_ref,
              w2_ref, b2_ref, g2_ref, be2_ref, out_ref,
              s1_ref, q1_ref, s2_ref, q2_ref):
    i = pl.program_id(0)
    phase = i // _NCHUNK

    def pre1():
        w1 = w1_ref[...]
        bc = lax.dot_general(nxp_ref[...], w1, _CDIM,
                             preferred_element_type=jnp.float32)
        p = lax.dot_general(g_ref[...], w1, _CDIM,
                            preferred_element_type=jnp.float32) + b1_ref[...]
        return (p.reshape(_GCHUNK, NSAMP, C1) - bc[:, None, :]).reshape(
            _CHUNK, C1)

    def hval():
        mu = s1_ref[...] / M
        var = q1_ref[...] / M - mu * mu
        sc = g1_ref[...] * lax.rsqrt(var + EPS)
        sh = be1_ref[...] - mu * sc
        return jnp.maximum(pre1() * sc + sh, 0.0)

    @pl.when(phase == 0)
    def _():
        p = pre1()

        @pl.when(i == 0)
        def _():
            s1_ref[...] = jnp.zeros((1, C1), jnp.float32)
            q1_ref[...] = jnp.zeros((1, C1), jnp.float32)

        s1_ref[...] += jnp.sum(p, axis=0, keepdims=True)
        q1_ref[...] += jnp.sum(p * p, axis=0, keepdims=True)

    @pl.when(phase == 1)
    def _():
        p2 = lax.dot_general(hval(), w2_ref[...], _CDIM,
                             preferred_element_type=jnp.float32) + b2_ref[...]

        @pl.when(i == _NCHUNK)
        def _():
            s2_ref[...] = jnp.zeros((1, C2), jnp.float32)
            q2_ref[...] = jnp.zeros((1, C2), jnp.float32)

        s2_ref[...] += jnp.sum(p2, axis=0, keepdims=True)
        q2_ref[...] += jnp.sum(p2 * p2, axis=0, keepdims=True)

    @pl.when(phase == 2)
    def _():
        mu = s2_ref[...] / M
        var = q2_ref[...] / M - mu * mu
        sc = g2_ref[...] * lax.rsqrt(var + EPS)
        sh = be2_ref[...] - mu * sc
        p2 = lax.dot_general(hval(), w2_ref[...], _CDIM,
                             preferred_element_type=jnp.float32) + b2_ref[...]
        y = jnp.maximum(p2 * sc + sh, 0.0)
        out_ref[...] = jnp.max(y.reshape(_GCHUNK, NSAMP, C2), axis=1)


def _mlp(g, nxp, w1, b1, g1, be1, w2, b2, g2, be2):
    nc = _NCHUNK

    def gmap(i):
        return (i % nc, 0)

    def omap(i):
        return (jnp.where(i >= 2 * nc, i - 2 * nc, 0), 0)

    def cmap(i):
        return (0, 0)

    return pl.pallas_call(
        _mlp_body,
        grid=(3 * nc,),
        in_specs=[pl.BlockSpec((_CHUNK, CH_IN), gmap),
                  pl.BlockSpec((_GCHUNK, CH_IN), gmap),
                  pl.BlockSpec((C1, CH_IN), cmap),
                  pl.BlockSpec((1, C1), cmap),
                  pl.BlockSpec((1, C1), cmap),
                  pl.BlockSpec((1, C1), cmap),
                  pl.BlockSpec((C2, C1), cmap),
                  pl.BlockSpec((1, C2), cmap),
                  pl.BlockSpec((1, C2), cmap),
                  pl.BlockSpec((1, C2), cmap)],
        out_specs=pl.BlockSpec((_GCHUNK, C2), omap),
        out_shape=jax.ShapeDtypeStruct((B * S, C2), jnp.float32),
        scratch_shapes=[
            pltpu.VMEM((1, C1), jnp.float32),
            pltpu.VMEM((1, C1), jnp.float32),
            pltpu.VMEM((1, C2), jnp.float32),
            pltpu.VMEM((1, C2), jnp.float32),
        ],
    )(g, nxp, w1, b1, g1, be1, w2, b2, g2, be2)


def kernel(xyz, features, W1, b1, g1, be1, W2, b2, g2, be2):
    xyzs = jnp.transpose(xyz, (2, 0, 1)).reshape(3 * B, N)
    xs = xyzs[0:B]
    ys = xyzs[B:2 * B]
    zs = xyzs[2 * B:3 * B]

    nx, ny, nz = _fps(xyzs)
    new_xyz = jnp.stack([nx, ny, nz], axis=-1)

    table = jnp.concatenate(
        [xyz, jnp.transpose(features, (0, 2, 1)),
         jnp.zeros((B, N, CH_IN - 3 - C), jnp.float32)], axis=-1,
    ).reshape(B * N, CH_IN)

    g = _ball_gather()(xs, ys, zs, nx, ny, nz, table)

    nxp = jnp.concatenate(
        [new_xyz.reshape(B * S, 3),
         jnp.zeros((B * S, CH_IN - 3), jnp.float32)], axis=-1)
    w1e = jnp.concatenate(
        [W1, jnp.zeros((C1, CH_IN - W1.shape[1]), jnp.float32)], axis=-1)

    out = _mlp(g, nxp, w1e, b1.reshape(1, C1), g1.reshape(1, C1),
               be1.reshape(1, C1), W2, b2.reshape(1, C2),
               g2.reshape(1, C2), be2.reshape(1, C2))

    new_features = jnp.transpose(out.reshape(B, S, C2), (0, 2, 1))
    return new_xyz, new_features
